# Initial kernel scaffold; baseline (speedup 1.0000x reference)
import functools

import jax
import jax.numpy as jnp
import numpy as np
from jax import lax
from jax.experimental import pallas as pl
from jax.experimental.pallas import tpu as pltpu
from jax.experimental.pallas import tpu_sc as plsc

_NF4 = np.array([
    -1.0, -0.6961928009986877, -0.5250730514526367, -0.39491748809814453,
    -0.28444138169288635, -0.18477343022823334, -0.09105003625154495, 0.0,
    0.07958029955625534, 0.16093020141124725, 0.24611230194568634,
    0.33791524171829224, 0.44070982933044434, 0.5626170039176941,
    0.7229568362236023, 1.0], dtype=np.float32)
_BOUND = ((_NF4[1:] + _NF4[:-1]) / 2.0).astype(np.float32)

_NBINS = 64
_HALF = _NBINS // 2


def _build_tables():
    tb = np.zeros(_NBINS, np.float32)
    lo = np.zeros(_NBINS, np.float32)
    hi = np.zeros(_NBINS, np.float32)
    for b in range(_NBINS):
        e0, e1 = -1.0 + b / _HALF, -1.0 + (b + 1) / _HALF
        inside = [j for j, bd in enumerate(_BOUND) if e0 < float(bd) < e1]
        assert len(inside) <= 1
        if inside:
            j = inside[0]
            tb[b] = _BOUND[j]
            lo[b] = _NF4[j]
            hi[b] = _NF4[j + 1]
        else:
            j = int(np.searchsorted(_BOUND, (e0 + e1) / 2.0, side='left'))
            tb[b] = 0.0
            lo[b] = _NF4[j]
            hi[b] = _NF4[j]
    return tb, lo, hi


_TB, _TLO, _THI = (jnp.asarray(t) for t in _build_tables())

_info = plsc.get_sparse_core_info()
_NC, _NS, _L = _info.num_cores, _info.num_subcores, _info.num_lanes
_NW = _NC * _NS

_N = 16 * 1024 * 768
_PER_W = _N // _NW
_CHUNK = 32768
_NCHUNK = _PER_W // _CHUNK
_VREGS = _CHUNK // _L


@functools.partial(
    pl.kernel,
    out_type=jax.ShapeDtypeStruct((_N,), jnp.float32),
    mesh=plsc.VectorSubcoreMesh(core_axis_name="c", subcore_axis_name="s"),
    scratch_types=[
        pltpu.VMEM((_CHUNK,), jnp.float32),
        pltpu.VMEM((_CHUNK,), jnp.float32),
        pltpu.VMEM((_NBINS,), jnp.float32),
        pltpu.VMEM((_NBINS,), jnp.float32),
        pltpu.VMEM((_NBINS,), jnp.float32),
        pltpu.VMEM((_L,), jnp.float32),
    ],
)
def _nf4_sc(x_hbm, tb_hbm, tlo_hbm, thi_hbm, a_hbm, out_hbm,
            inb, outb, tb_v, tlo_v, thi_v, a_v):
    wid = lax.axis_index("s") * _NC + lax.axis_index("c")
    base = wid * _PER_W
    pltpu.sync_copy(tb_hbm, tb_v)
    pltpu.sync_copy(tlo_hbm, tlo_v)
    pltpu.sync_copy(thi_hbm, thi_v)
    pltpu.sync_copy(a_hbm, a_v)
    a = a_v[...]
    off_c = jnp.full((_L,), float(_HALF), jnp.float32)
    zero_c = jnp.zeros((_L,), jnp.float32)
    top_c = jnp.full((_L,), float(_NBINS - 1), jnp.float32)

    def chunk_body(c, carry):
        off = base + c * _CHUNK
        pltpu.sync_copy(x_hbm.at[pl.ds(off, _CHUNK)], inb)

        def vreg_body(i, carry2):
            v = inb[pl.ds(i * _L, _L)]
            f = v * a + off_c
            f = jnp.minimum(jnp.maximum(f, zero_c), top_c)
            idx = f.astype(jnp.int32)
            bnd = plsc.load_gather(tb_v, [idx])
            lov = plsc.load_gather(tlo_v, [idx])
            hiv = plsc.load_gather(thi_v, [idx])
            outb[pl.ds(i * _L, _L)] = jnp.where(v > bnd, hiv, lov)
            return carry2

        lax.fori_loop(0, _VREGS, vreg_body, 0)
        pltpu.sync_copy(outb, out_hbm.at[pl.ds(off, _CHUNK)])
        return carry

    lax.fori_loop(0, _NCHUNK, chunk_body, 0)


def kernel(x, scale):
    s = jnp.clip(scale.astype(jnp.float32), 1e-8, None)
    thr = _TB * s
    neg_inf = jnp.float32(-jnp.inf)
    pos_inf = jnp.float32(jnp.inf)
    for _ in range(3):
        thr = jnp.where(thr / s > _TB, jnp.nextafter(thr, neg_inf), thr)
    for _ in range(3):
        up = jnp.nextafter(thr, pos_inf)
        thr = jnp.where(up / s <= _TB, up, thr)
    lo_s = _TLO * s
    hi_s = _THI * s
    a_vec = jnp.broadcast_to(jnp.float32(_HALF) / s, (_L,))
    y = _nf4_sc(x.reshape(-1), thr, lo_s, hi_s, a_vec)
    return y.reshape(x.shape)


# speedup vs baseline: 10131.5029x; 10131.5029x over previous
import functools

import jax
import jax.numpy as jnp
import numpy as np
from jax import lax
from jax.experimental import pallas as pl
from jax.experimental.pallas import tpu as pltpu
from jax.experimental.pallas import tpu_sc as plsc

_NF4 = np.array([
    -1.0, -0.6961928009986877, -0.5250730514526367, -0.39491748809814453,
    -0.28444138169288635, -0.18477343022823334, -0.09105003625154495, 0.0,
    0.07958029955625534, 0.16093020141124725, 0.24611230194568634,
    0.33791524171829224, 0.44070982933044434, 0.5626170039176941,
    0.7229568362236023, 1.0], dtype=np.float32)
_BOUND = ((_NF4[1:] + _NF4[:-1]) / 2.0).astype(np.float32)

_NBINS = 64
_HALF = _NBINS // 2


def _build_tables():
    tb = np.zeros(_NBINS, np.float32)
    lo = np.zeros(_NBINS, np.float32)
    hi = np.zeros(_NBINS, np.float32)
    for b in range(_NBINS):
        e0, e1 = -1.0 + b / _HALF, -1.0 + (b + 1) / _HALF
        inside = [j for j, bd in enumerate(_BOUND) if e0 < float(bd) < e1]
        assert len(inside) <= 1
        if inside:
            j = inside[0]
            tb[b] = _BOUND[j]
            lo[b] = _NF4[j]
            hi[b] = _NF4[j + 1]
        else:
            j = int(np.searchsorted(_BOUND, (e0 + e1) / 2.0, side='left'))
            tb[b] = 0.0
            lo[b] = _NF4[j]
            hi[b] = _NF4[j]
    return tb, lo, hi


_TB_NP, _TLO_NP, _THI_NP = _build_tables()

_NC, _NS, _L = 2, 16, 16
_NW = _NC * _NS

_N = 16 * 1024 * 768
_CHUNK = 24576
_UNROLL = 8
_VREGS = _CHUNK // _L

_SC_CHUNKS = 0
_NSC = _NW * _SC_CHUNKS * _CHUNK

_COLS = 768
_TC_BR = 512


@functools.cache
def _nf4_sc(nchunks):
    per_w = nchunks * _CHUNK

    @functools.partial(
        pl.kernel,
        out_type=jax.ShapeDtypeStruct((_NW * per_w,), jnp.float32),
        mesh=plsc.VectorSubcoreMesh(core_axis_name="c", subcore_axis_name="s"),
        compiler_params=pltpu.CompilerParams(needs_layout_passes=False),
        scratch_types=[
            pltpu.VMEM((_CHUNK,), jnp.float32),
            pltpu.VMEM((_CHUNK,), jnp.float32),
            pltpu.VMEM((_CHUNK,), jnp.float32),
            pltpu.VMEM((_CHUNK,), jnp.float32),
            pltpu.VMEM((_NBINS,), jnp.float32),
            pltpu.VMEM((2 * _NBINS,), jnp.float32),
            pltpu.VMEM((_L,), jnp.float32),
            pltpu.SemaphoreType.DMA,
            pltpu.SemaphoreType.DMA,
            pltpu.SemaphoreType.DMA,
            pltpu.SemaphoreType.DMA,
        ],
    )
    def body(x_hbm, tb_hbm, vt_hbm, a_hbm, out_hbm,
             in0, in1, out0, out1, tb_v, vt_v, a_v,
             si0, si1, so0, so1):
        wid = lax.axis_index("s") * _NC + lax.axis_index("c")
        base = wid * per_w
        pltpu.sync_copy(tb_hbm, tb_v)
        pltpu.sync_copy(vt_hbm, vt_v)
        pltpu.sync_copy(a_hbm, a_v)
        a = a_v[...]
        off_c = jnp.full((_L,), float(_HALF), jnp.float32)
        zero_c = jnp.zeros((_L,), jnp.float32)
        top_c = jnp.full((_L,), float(_NBINS - 1), jnp.float32)
        one_i = jnp.ones((_L,), jnp.int32)
        zero_i = jnp.zeros((_L,), jnp.int32)

        def compute(inb, outb):
            @plsc.parallel_loop(0, _VREGS, unroll=_UNROLL)
            def vreg_body(i):
                v = inb[pl.ds(i * _L, _L)]
                f = v * a + off_c
                f = jnp.minimum(jnp.maximum(f, zero_c), top_c)
                idx = f.astype(jnp.int32)
                bnd = plsc.load_gather(tb_v, [idx])
                up = jnp.where(v > bnd, one_i, zero_i)
                outb[pl.ds(i * _L, _L)] = plsc.load_gather(
                    vt_v, [idx + idx + up])

        def wait_in(buf, sem):
            pltpu.make_async_copy(x_hbm.at[pl.ds(0, _CHUNK)], buf, sem).wait()

        def wait_out(buf, sem):
            pltpu.make_async_copy(buf, out_hbm.at[pl.ds(0, _CHUNK)], sem).wait()

        pltpu.async_copy(x_hbm.at[pl.ds(base, _CHUNK)], in0, si0)

        def pair_body(p, carry):
            c0 = base + 2 * p * _CHUNK
            c1 = c0 + _CHUNK
            pltpu.async_copy(x_hbm.at[pl.ds(c1, _CHUNK)], in1, si1)
            wait_in(in0, si0)

            @pl.when(p > 0)
            def _():
                wait_out(out0, so0)

            compute(in0, out0)
            pltpu.async_copy(out0, out_hbm.at[pl.ds(c0, _CHUNK)], so0)

            @pl.when(p < nchunks // 2 - 1)
            def _():
                pltpu.async_copy(
                    x_hbm.at[pl.ds(c1 + _CHUNK, _CHUNK)], in0, si0)

            wait_in(in1, si1)

            @pl.when(p > 0)
            def _():
                wait_out(out1, so1)

            compute(in1, out1)
            pltpu.async_copy(out1, out_hbm.at[pl.ds(c1, _CHUNK)], so1)
            return carry

        lax.fori_loop(0, nchunks // 2, pair_body, 0)
        wait_out(out0, so0)
        wait_out(out1, so1)

    return body


def _tc_body(x_ref, thr_ref, val_ref, o_ref):
    v = x_ref[...]
    r = jnp.zeros_like(v) + val_ref[0]
    for j in range(15):
        r = jnp.where(v > thr_ref[j], val_ref[j + 1], r)
    o_ref[...] = r


@functools.cache
def _nf4_tc(rows):
    return pl.pallas_call(
        _tc_body,
        grid=(rows // _TC_BR,),
        in_specs=[
            pl.BlockSpec((_TC_BR, _COLS), lambda i: (i, 0)),
            pl.BlockSpec(memory_space=pltpu.SMEM),
            pl.BlockSpec(memory_space=pltpu.SMEM),
        ],
        out_specs=pl.BlockSpec((_TC_BR, _COLS), lambda i: (i, 0)),
        out_shape=jax.ShapeDtypeStruct((rows, _COLS), jnp.float32),
    )


def _exact_thresholds(raw, s):
    thr = raw * s
    neg_inf = jnp.float32(-jnp.inf)
    pos_inf = jnp.float32(jnp.inf)
    for _ in range(3):
        thr = jnp.where(thr / s > raw, jnp.nextafter(thr, neg_inf), thr)
    for _ in range(3):
        up = jnp.nextafter(thr, pos_inf)
        thr = jnp.where(up / s <= raw, up, thr)
    return thr


def kernel(x, scale):
    s = jnp.clip(scale.astype(jnp.float32), 1e-8, None)
    xf = x.reshape(-1)
    parts = []
    if _SC_CHUNKS:
        tb = jnp.asarray(_TB_NP)
        thr_bins = _exact_thresholds(tb, s)
        vt = (jnp.stack([jnp.asarray(_TLO_NP), jnp.asarray(_THI_NP)], axis=1)
              * s).reshape(-1)
        a_vec = jnp.broadcast_to(jnp.float32(_HALF) / s, (_L,))
        parts.append(_nf4_sc(_SC_CHUNKS)(xf[:_NSC], thr_bins, vt, a_vec))
    if _NSC < _N:
        thr_full = _exact_thresholds(jnp.asarray(_BOUND), s)
        vals = jnp.asarray(_NF4) * s
        rows = (_N - _NSC) // _COLS
        ytc = _nf4_tc(rows)(xf[_NSC:].reshape(rows, _COLS), thr_full, vals)
        parts.append(ytc.reshape(-1))
    y = parts[0] if len(parts) == 1 else jnp.concatenate(parts)
    return y.reshape(x.shape)
